# baseline (speedup 1.0000x reference)
import functools
import jax
import jax.numpy as jnp
from jax import lax
from jax.experimental import pallas as pl
from jax.experimental.pallas import tpu as pltpu
from jax.experimental.pallas import tpu_sc as plsc

B, L = 4096, 200
NUCL_DIM, SPEC_DIM = 128, 32
OUT_DIM = NUCL_DIM + 2 * SPEC_DIM
NROWS = B * L
TBL = 32

NC, NS = 2, 16
NW = NC * NS
ROWS_PER_W = NROWS // NW
CH = 128
NCH = ROWS_PER_W // CH


def _prep_body(x_ref, pbs_ref, rt_ref, nucl_ref, pbst_ref, rtt_ref,
               fidx_ref, ntbl_ref, stbl_ref):
    p = (pbs_ref[...] > 0.5).astype(jnp.int32)
    r = (rt_ref[...] > 0.5).astype(jnp.int32)
    q = p * 2 + r
    fidx_ref[...] = q * 8 + x_ref[...]

    c = lax.broadcasted_iota(jnp.int32, (TBL, NUCL_DIM), 0)
    xt = c % 8
    n = jnp.zeros((TBL, NUCL_DIM), jnp.float32)
    for v in range(5):
        n = jnp.where(xt == v, nucl_ref[v, :][None, :], n)
    ntbl_ref[...] = n

    cs = lax.broadcasted_iota(jnp.int32, (TBL, SPEC_DIM), 0)
    pb = jnp.where((cs // 8) // 2 == 1, pbst_ref[1, :][None, :],
                   pbst_ref[0, :][None, :])
    rb = jnp.where((cs // 8) % 2 == 1, rtt_ref[1, :][None, :],
                   rtt_ref[0, :][None, :])
    z = jnp.zeros((TBL, NUCL_DIM - 2 * SPEC_DIM), jnp.float32)
    stbl_ref[...] = jnp.concatenate([pb, rb, z], axis=1)


def _prep(x_seq, pbs_feat, rt_feat, nucl_table, pbs_table, rt_table):
    return pl.pallas_call(
        _prep_body,
        in_specs=[
            pl.BlockSpec((B, L), lambda: (0, 0)),
            pl.BlockSpec((B, 1), lambda: (0, 0)),
            pl.BlockSpec((B, 1), lambda: (0, 0)),
            pl.BlockSpec((5, NUCL_DIM), lambda: (0, 0)),
            pl.BlockSpec((2, SPEC_DIM), lambda: (0, 0)),
            pl.BlockSpec((2, SPEC_DIM), lambda: (0, 0)),
        ],
        out_specs=[
            pl.BlockSpec((B, L), lambda: (0, 0)),
            pl.BlockSpec((TBL, NUCL_DIM), lambda: (0, 0)),
            pl.BlockSpec((TBL, NUCL_DIM), lambda: (0, 0)),
        ],
        out_shape=[
            jax.ShapeDtypeStruct((B, L), jnp.int32),
            jax.ShapeDtypeStruct((TBL, NUCL_DIM), jnp.float32),
            jax.ShapeDtypeStruct((TBL, NUCL_DIM), jnp.float32),
        ],
    )(x_seq, pbs_feat.reshape(B, 1), rt_feat.reshape(B, 1),
      nucl_table, pbs_table, rt_table)


@functools.partial(
    pl.kernel,
    out_type=jax.ShapeDtypeStruct((NROWS, OUT_DIM), jnp.float32),
    mesh=plsc.VectorSubcoreMesh(core_axis_name="c", subcore_axis_name="s"),
    scratch_types=[
        pltpu.VMEM((NCH, CH), jnp.int32),
        pltpu.VMEM((CH, OUT_DIM), jnp.float32),
        pltpu.VMEM((CH, OUT_DIM), jnp.float32),
        pltpu.VMEM((CH, NUCL_DIM), jnp.float32),
        pltpu.VMEM((CH, NUCL_DIM), jnp.float32),
        pltpu.SemaphoreType.DMA,
        pltpu.SemaphoreType.DMA,
        pltpu.SemaphoreType.DMA,
        pltpu.SemaphoreType.DMA,
        pltpu.SemaphoreType.DMA,
        pltpu.SemaphoreType.DMA,
    ],
)
def _sc_gather(fidx_hbm, ntbl_hbm, stbl_hbm, out_hbm,
               idx_all, comb_a, comb_b, spec_a, spec_b,
               gn_a, gn_b, gs_a, gs_b, ws_a, ws_b):
    wid = lax.axis_index("s") * NC + lax.axis_index("c")
    base0 = wid * ROWS_PER_W
    pltpu.sync_copy(fidx_hbm.at[wid], idx_all)

    combs = (comb_a, comb_b)
    specs = (spec_a, spec_b)
    gnsems = (gn_a, gn_b)
    gssems = (gs_a, gs_b)
    wsems = (ws_a, ws_b)

    def g_start(i, b):
        pltpu.async_copy(ntbl_hbm.at[idx_all.at[i]],
                         combs[b].at[:, pl.ds(0, NUCL_DIM)], gnsems[b])
        pltpu.async_copy(stbl_hbm.at[idx_all.at[i]], specs[b], gssems[b])

    def g_wait(b):
        pltpu.make_async_copy(ntbl_hbm.at[idx_all.at[0]],
                              combs[b].at[:, pl.ds(0, NUCL_DIM)],
                              gnsems[b]).wait()
        pltpu.make_async_copy(stbl_hbm.at[idx_all.at[0]], specs[b],
                              gssems[b]).wait()

    def w_start(i, b):
        pltpu.async_copy(combs[b], out_hbm.at[pl.ds(base0 + i * CH, CH)],
                         wsems[b])

    def w_wait(b):
        pltpu.make_async_copy(combs[b], out_hbm.at[pl.ds(0, CH)],
                              wsems[b]).wait()

    def spec_fill(b):
        comb, spec = combs[b], specs[b]

        def grp(g, carry):
            r0 = g * 8
            for k in range(8):
                for m in range(4):
                    comb[r0 + k, pl.ds(NUCL_DIM + 16 * m, 16)] = \
                        spec[r0 + k, pl.ds(16 * m, 16)]
            return carry

        lax.fori_loop(0, CH // 8, grp, 0)

    g_start(0, 0)
    g_wait(0)
    g_start(1, 1)
    spec_fill(0)
    w_start(0, 0)

    def body(j, carry):
        for b in (1, 0):
            i = 2 * j + (1 if b == 1 else 2)
            g_wait(b)
            w_wait(1 - b)
            g_start(i + 1, 1 - b)
            spec_fill(b)
            w_start(i, b)
        return carry

    lax.fori_loop(0, (NCH - 2) // 2, body, 0)

    g_wait(1)
    spec_fill(1)
    w_start(NCH - 1, 1)
    w_wait(0)
    w_wait(1)


@jax.jit
def kernel(x_seq, pbs_feat, rt_feat, nucl_table, pbs_table, rt_table):
    fidx, ntbl, stbl = _prep(x_seq, pbs_feat, rt_feat,
                             nucl_table, pbs_table, rt_table)
    out = _sc_gather(fidx.reshape(NW, NCH, CH), ntbl, stbl)
    return out.reshape(B, L, OUT_DIM)


# speedup vs baseline: 2.0421x; 2.0421x over previous
import functools
import jax
import jax.numpy as jnp
from jax import lax
from jax.experimental import pallas as pl
from jax.experimental.pallas import tpu as pltpu
from jax.experimental.pallas import tpu_sc as plsc

B, L = 4096, 200
NUCL_DIM, SPEC_DIM = 128, 32
OUT_DIM = NUCL_DIM + 2 * SPEC_DIM
QUAD_DIM = 4 * OUT_DIM
NQUAD = B * (L // 4)
TBL = 2560

NC, NS = 2, 16
NW = NC * NS
ROWS_PER_W = NQUAD // NW
CH = 64
NCH = ROWS_PER_W // CH


def _prep_body(x_ref, pbs_ref, rt_ref, nucl_ref, pbst_ref, rtt_ref,
               qidx_ref, tbl_ref):
    p = (pbs_ref[...] > 0.5).astype(jnp.int32)
    r = (rt_ref[...] > 0.5).astype(jnp.int32)
    q = p * 2 + r

    m = lax.broadcasted_iota(jnp.int32, (B, L), 1) % 4
    w = jnp.where(m == 0, 500.0,
                  jnp.where(m == 1, 100.0,
                            jnp.where(m == 2, 20.0, 4.0)))
    y = x_ref[...].astype(jnp.float32) * w
    li = lax.broadcasted_iota(jnp.int32, (L, L // 4), 0)
    ji = lax.broadcasted_iota(jnp.int32, (L, L // 4), 1)
    s = (li // 4 == ji).astype(jnp.float32)
    acc = lax.dot_general(y, s, (((1,), (0,)), ((), ())),
                          precision=lax.Precision.HIGHEST,
                          preferred_element_type=jnp.float32)
    qidx_ref[...] = acc.astype(jnp.int32) + q

    c = lax.broadcasted_iota(jnp.int32, (TBL, NUCL_DIM), 0)
    xs = [c // 500, (c // 100) % 5, (c // 20) % 5, (c // 4) % 5]
    ns = []
    for k in range(4):
        n = jnp.zeros((TBL, NUCL_DIM), jnp.float32)
        for v in range(5):
            n = jnp.where(xs[k] == v, nucl_ref[v, :][None, :], n)
        ns.append(n)
    cs = lax.broadcasted_iota(jnp.int32, (TBL, SPEC_DIM), 0)
    pb = jnp.where((cs % 4) // 2 == 1, pbst_ref[1, :][None, :],
                   pbst_ref[0, :][None, :])
    rb = jnp.where(cs % 2 == 1, rtt_ref[1, :][None, :],
                   rtt_ref[0, :][None, :])
    tbl_ref[...] = jnp.concatenate(
        [ns[0], pb, rb, ns[1], pb, rb, ns[2], pb, rb, ns[3], pb, rb], axis=1)


def _prep(x_seq, pbs_feat, rt_feat, nucl_table, pbs_table, rt_table):
    return pl.pallas_call(
        _prep_body,
        in_specs=[
            pl.BlockSpec((B, L), lambda: (0, 0)),
            pl.BlockSpec((B, 1), lambda: (0, 0)),
            pl.BlockSpec((B, 1), lambda: (0, 0)),
            pl.BlockSpec((5, NUCL_DIM), lambda: (0, 0)),
            pl.BlockSpec((2, SPEC_DIM), lambda: (0, 0)),
            pl.BlockSpec((2, SPEC_DIM), lambda: (0, 0)),
        ],
        out_specs=[
            pl.BlockSpec((B, L // 4), lambda: (0, 0)),
            pl.BlockSpec((TBL, QUAD_DIM), lambda: (0, 0)),
        ],
        out_shape=[
            jax.ShapeDtypeStruct((B, L // 4), jnp.int32),
            jax.ShapeDtypeStruct((TBL, QUAD_DIM), jnp.float32),
        ],
    )(x_seq, pbs_feat.reshape(B, 1), rt_feat.reshape(B, 1),
      nucl_table, pbs_table, rt_table)


@functools.partial(
    pl.kernel,
    out_type=jax.ShapeDtypeStruct((NQUAD, QUAD_DIM), jnp.float32),
    mesh=plsc.VectorSubcoreMesh(core_axis_name="c", subcore_axis_name="s"),
    scratch_types=[
        pltpu.VMEM((NCH, CH), jnp.int32),
        pltpu.VMEM((CH, QUAD_DIM), jnp.float32),
        pltpu.VMEM((CH, QUAD_DIM), jnp.float32),
        pltpu.SemaphoreType.DMA,
        pltpu.SemaphoreType.DMA,
        pltpu.SemaphoreType.DMA,
        pltpu.SemaphoreType.DMA,
    ],
)
def _sc_gather(qidx_hbm, tbl_hbm, out_hbm,
               idx_all, buf_a, buf_b, gs_a, gs_b, ws_a, ws_b):
    wid = lax.axis_index("s") * NC + lax.axis_index("c")
    base0 = wid * ROWS_PER_W
    pltpu.sync_copy(qidx_hbm.at[wid], idx_all)

    bufs = (buf_a, buf_b)
    gsems = (gs_a, gs_b)
    wsems = (ws_a, ws_b)

    def g_start(i, b):
        pltpu.async_copy(tbl_hbm.at[idx_all.at[i]], bufs[b], gsems[b])

    def g_wait(b):
        pltpu.make_async_copy(tbl_hbm.at[idx_all.at[0]], bufs[b],
                              gsems[b]).wait()

    def w_start(i, b):
        pltpu.async_copy(bufs[b], out_hbm.at[pl.ds(base0 + i * CH, CH)],
                         wsems[b])

    def w_wait(b):
        pltpu.make_async_copy(bufs[b], out_hbm.at[pl.ds(0, CH)],
                              wsems[b]).wait()

    g_start(0, 0)
    g_wait(0)
    g_start(1, 1)
    w_start(0, 0)

    def body(j, carry):
        for b in (1, 0):
            i = 2 * j + (1 if b == 1 else 2)
            g_wait(b)
            w_wait(1 - b)
            g_start(i + 1, 1 - b)
            w_start(i, b)
        return carry

    lax.fori_loop(0, (NCH - 2) // 2, body, 0)

    g_wait(1)
    w_start(NCH - 1, 1)
    w_wait(0)
    w_wait(1)


@jax.jit
def kernel(x_seq, pbs_feat, rt_feat, nucl_table, pbs_table, rt_table):
    qidx, tbl = _prep(x_seq, pbs_feat, rt_feat,
                      nucl_table, pbs_table, rt_table)
    out = _sc_gather(qidx.reshape(NW, NCH, CH), tbl)
    return out.reshape(B, L, OUT_DIM)
